# baseline (speedup 1.0000x reference)
import functools

import jax
import jax.numpy as jnp
from jax import lax
from jax.experimental import pallas as pl
from jax.experimental.pallas import tpu as pltpu
from jax.experimental.pallas import tpu_sc as plsc

N_NODES = 100000
N_EDGES = 3200000
D_NODE = 128
D_EDGE = 16
LATENT = 32
D_OUT = 128

NC = 2
NS = 16
N_CT = N_EDGES // 128
CT_H = N_CT // NC
K = 25
N_CHK = CT_H // K
NB = 4
N_PAD = 100096
assert CT_H % K == 0 and N_CHK % NB == 0


def _sc_partials(ea4, ei3):
    mesh = plsc.VectorSubcoreMesh(core_axis_name="c", subcore_axis_name="s")

    @functools.partial(
        pl.kernel,
        out_type=jax.ShapeDtypeStruct((NC, NS, N_PAD), jnp.float32),
        mesh=mesh,
        scratch_types=[
            pltpu.VMEM((NB, K, 128), jnp.int32),
            pltpu.VMEM((NB, K, 128), jnp.float32),
            pltpu.SemaphoreType.DMA,
            pltpu.SemaphoreType.DMA,
            pltpu.SemaphoreType.DMA,
            pltpu.SemaphoreType.DMA,
            pltpu.VMEM((N_PAD,), jnp.float32),
        ],
        compiler_params=pltpu.CompilerParams(use_tc_tiling_on_sc=False,
                                             needs_layout_passes=False),
    )
    def sc_kernel(ea_hbm, ei_hbm, out_hbm, idx_v, col_v, sem0, sem1, sem2,
                  sem3, acc):
        cid = lax.axis_index("c")
        sid = lax.axis_index("s")
        sems = [sem0, sem1, sem2, sem3]
        a = sid // 8
        s = sid % 8
        ct_base = cid * CT_H

        def zero_body(i, carry):
            acc[pl.ds(i * 16, 16)] = jnp.zeros((16,), jnp.float32)
            return carry

        lax.fori_loop(0, N_PAD // 16, zero_body, 0)

        def start_load(c, b):
            ct0 = ct_base + c * K
            pltpu.async_copy(ei_hbm.at[pl.ds(ct0, K), 1, :], idx_v.at[b],
                             sems[b])
            pltpu.async_copy(ea_hbm.at[a, pl.ds(ct0, K), s, :], col_v.at[b],
                             sems[b])

        def wait_load(c, b):
            ct0 = ct_base + c * K
            pltpu.make_async_copy(ei_hbm.at[pl.ds(ct0, K), 1, :],
                                  idx_v.at[b], sems[b]).wait()
            pltpu.make_async_copy(ea_hbm.at[a, pl.ds(ct0, K), s, :],
                                  col_v.at[b], sems[b]).wait()

        for b0 in range(NB - 1):
            start_load(b0, b0)

        def chunk_body(t, carry):
            for b in range(NB):
                c = NB * t + b

                @pl.when(c + NB - 1 < N_CHK)
                def _():
                    start_load(c + NB - 1, (b + NB - 1) % NB)

                wait_load(c, b)
                for k in range(K):
                    for g in range(8):
                        iv = idx_v[b, k, pl.ds(g * 16, 16)]
                        xv = col_v[b, k, pl.ds(g * 16, 16)]
                        plsc.addupdate_scatter(acc, [iv], xv)
            return carry

        lax.fori_loop(0, N_CHK // NB, chunk_body, 0)

        pltpu.sync_copy(acc, out_hbm.at[cid, sid])

    return sc_kernel(ea4, ei3)


def _mlp_body(x_ref, pa_ref, pb_ref, g_ref, w1_ref, b1_ref, w2_ref, b2_ref,
              out_ref):
    aggt = pa_ref[0] + pb_ref[0]
    w1 = w1_ref[...]
    c = jnp.dot(g_ref[...], w1[D_NODE + D_EDGE:, :],
                preferred_element_type=jnp.float32) + b1_ref[...]
    h = (jnp.dot(x_ref[...], w1[:D_NODE, :], preferred_element_type=jnp.float32)
         + lax.dot_general(aggt, w1[D_NODE:D_NODE + D_EDGE, :],
                           ((( 0,), (0,)), ((), ())),
                           preferred_element_type=jnp.float32)
         + c)
    h = jnp.maximum(h, 0.0)
    out_ref[...] = jnp.dot(h, w2_ref[...],
                           preferred_element_type=jnp.float32) + b2_ref[...]


def _mlp(x, parts, global_attr, W1, b1, W2, b2):
    RB = 2048
    grid = ((N_NODES + RB - 1) // RB,)
    return pl.pallas_call(
        _mlp_body,
        grid=grid,
        in_specs=[
            pl.BlockSpec((RB, D_NODE), lambda i: (i, 0)),
            pl.BlockSpec((1, NS, RB), lambda i: (0, 0, i)),
            pl.BlockSpec((1, NS, RB), lambda i: (1, 0, i)),
            pl.BlockSpec((1, D_EDGE), lambda i: (0, 0)),
            pl.BlockSpec((D_NODE + 2 * D_EDGE, LATENT), lambda i: (0, 0)),
            pl.BlockSpec((1, LATENT), lambda i: (0, 0)),
            pl.BlockSpec((LATENT, D_OUT), lambda i: (0, 0)),
            pl.BlockSpec((1, D_OUT), lambda i: (0, 0)),
        ],
        out_specs=pl.BlockSpec((RB, D_OUT), lambda i: (i, 0)),
        out_shape=jax.ShapeDtypeStruct((N_NODES, D_OUT), jnp.float32),
    )(x, parts, parts, global_attr, W1, b1, W2, b2)


def kernel(x, edge_index, edge_attr, global_attr, W1, b1, W2, b2):
    ea4 = edge_attr.T.reshape(2, 8, N_CT, 128).transpose(0, 2, 1, 3)
    ei3 = edge_index.astype(jnp.int32).T.reshape(N_CT, 128, 2).transpose(0, 2, 1)
    parts = _sc_partials(ea4, ei3)
    return _mlp(x, parts, global_attr, W1,
                b1.reshape(1, LATENT), W2, b2.reshape(1, D_OUT))


# speedup vs baseline: 1.1249x; 1.1249x over previous
import functools

import jax
import jax.numpy as jnp
from jax import lax
from jax.experimental import pallas as pl
from jax.experimental.pallas import tpu as pltpu
from jax.experimental.pallas import tpu_sc as plsc

N_NODES = 100000
N_EDGES = 3200000
D_NODE = 128
D_EDGE = 16
LATENT = 32
D_OUT = 128

NC = 2
NS = 16
N_CT = N_EDGES // 128
CT_H = N_CT // NC
K = 25
N_CHK = CT_H // K
NB = 2
N_PAD = 100096
assert CT_H % K == 0 and N_CHK % NB == 0


def _sc_partials(ea4, ei3):
    mesh = plsc.VectorSubcoreMesh(core_axis_name="c", subcore_axis_name="s")

    @functools.partial(
        pl.kernel,
        out_type=jax.ShapeDtypeStruct((NC, NS, N_PAD), jnp.float32),
        mesh=mesh,
        scratch_types=[
            pltpu.VMEM((NB, K, 128), jnp.int32),
            pltpu.VMEM((NB, K, 128), jnp.float32),
            pltpu.SemaphoreType.DMA,
            pltpu.SemaphoreType.DMA,
            pltpu.VMEM((N_PAD,), jnp.float32),
        ],
        compiler_params=pltpu.CompilerParams(use_tc_tiling_on_sc=False,
                                             needs_layout_passes=False),
    )
    def sc_kernel(ea_hbm, ei_hbm, out_hbm, idx_v, col_v, sem0, sem1, acc):
        cid = lax.axis_index("c")
        sid = lax.axis_index("s")
        sems = [sem0, sem1]
        a = sid // 8
        s = sid % 8
        ct_base = cid * CT_H

        def zero_body(i, carry):
            acc[pl.ds(i * 16, 16)] = jnp.zeros((16,), jnp.float32)
            return carry

        lax.fori_loop(0, N_PAD // 16, zero_body, 0)

        def start_load(c, b):
            ct0 = ct_base + c * K
            pltpu.async_copy(ei_hbm.at[pl.ds(ct0, K), 1, :], idx_v.at[b],
                             sems[b])
            pltpu.async_copy(ea_hbm.at[a, pl.ds(ct0, K), s, :], col_v.at[b],
                             sems[b])

        def wait_load(c, b):
            ct0 = ct_base + c * K
            pltpu.make_async_copy(ei_hbm.at[pl.ds(ct0, K), 1, :],
                                  idx_v.at[b], sems[b]).wait()
            pltpu.make_async_copy(ea_hbm.at[a, pl.ds(ct0, K), s, :],
                                  col_v.at[b], sems[b]).wait()

        for b0 in range(NB - 1):
            start_load(b0, b0)

        def chunk_body(t, carry):
            for b in range(NB):
                c = NB * t + b

                @pl.when(c + NB - 1 < N_CHK)
                def _():
                    start_load(c + NB - 1, (b + NB - 1) % NB)

                wait_load(c, b)
                for k in range(K):
                    for g in range(8):
                        iv = idx_v[b, k, pl.ds(g * 16, 16)]
                        xv = col_v[b, k, pl.ds(g * 16, 16)]
                        plsc.addupdate_scatter(acc, [iv], xv)
            return carry

        lax.fori_loop(0, N_CHK // NB, chunk_body, 0)

        pltpu.sync_copy(acc, out_hbm.at[cid, sid])

    return sc_kernel(ea4, ei3)


def _mlp_body(x_ref, pa_ref, pb_ref, g_ref, w1_ref, b1_ref, w2_ref, b2_ref,
              out_ref):
    aggt = pa_ref[0] + pb_ref[0]
    w1 = w1_ref[...]
    c = jnp.dot(g_ref[...], w1[D_NODE + D_EDGE:, :],
                preferred_element_type=jnp.float32) + b1_ref[...]
    h = (jnp.dot(x_ref[...], w1[:D_NODE, :], preferred_element_type=jnp.float32)
         + lax.dot_general(aggt, w1[D_NODE:D_NODE + D_EDGE, :],
                           ((( 0,), (0,)), ((), ())),
                           preferred_element_type=jnp.float32)
         + c)
    h = jnp.maximum(h, 0.0)
    out_ref[...] = jnp.dot(h, w2_ref[...],
                           preferred_element_type=jnp.float32) + b2_ref[...]


def _mlp(x, parts, global_attr, W1, b1, W2, b2):
    RB = 2048
    grid = ((N_NODES + RB - 1) // RB,)
    return pl.pallas_call(
        _mlp_body,
        grid=grid,
        in_specs=[
            pl.BlockSpec((RB, D_NODE), lambda i: (i, 0)),
            pl.BlockSpec((1, NS, RB), lambda i: (0, 0, i)),
            pl.BlockSpec((1, NS, RB), lambda i: (1, 0, i)),
            pl.BlockSpec((1, D_EDGE), lambda i: (0, 0)),
            pl.BlockSpec((D_NODE + 2 * D_EDGE, LATENT), lambda i: (0, 0)),
            pl.BlockSpec((1, LATENT), lambda i: (0, 0)),
            pl.BlockSpec((LATENT, D_OUT), lambda i: (0, 0)),
            pl.BlockSpec((1, D_OUT), lambda i: (0, 0)),
        ],
        out_specs=pl.BlockSpec((RB, D_OUT), lambda i: (i, 0)),
        out_shape=jax.ShapeDtypeStruct((N_NODES, D_OUT), jnp.float32),
    )(x, parts, parts, global_attr, W1, b1, W2, b2)


def kernel(x, edge_index, edge_attr, global_attr, W1, b1, W2, b2):
    ea4 = edge_attr.T.reshape(2, 8, N_CT, 128).transpose(0, 2, 1, 3)
    ei3 = edge_index.astype(jnp.int32).T.reshape(N_CT, 128, 2).transpose(0, 2, 1)
    parts = _sc_partials(ea4, ei3)
    return _mlp(x, parts, global_attr, W1,
                b1.reshape(1, LATENT), W2, b2.reshape(1, D_OUT))
